# Initial kernel scaffold; baseline (speedup 1.0000x reference)
import jax
import jax.numpy as jnp
from jax.experimental import pallas as pl


def kernel(sensor_seq, mask, node_emb, edge_src, edge_dst, W_in, W_msg, W_self, w_att, W_out, b_out):
    raise NotImplementedError("write your pallas kernel here")


# speedup vs baseline: 55.4026x; 55.4026x over previous
import functools

import jax
import jax.numpy as jnp
from jax.experimental import pallas as pl

C = 8
N_NODES = 32
D_HID = 128
E_EDGES = 128
D_IN = 1024
B = 32
T = 30
N_F = B * T
NODE_CHUNK = 4


def _branch_kernel(x_ref, emb_ref, es_row_ref, es_col_ref, ed_row_ref,
                   ed_col_ref, win_ref, wmsg_ref, wself_ref, watt_ref,
                   wout_ref, logits_ref, v_ref):
    c = pl.program_id(0)
    f32 = jnp.float32

    src_row = es_row_ref[0]
    src_col = es_col_ref[0]
    dst_row = ed_row_ref[0]
    dst_col = ed_col_ref[0]

    iota_ne = jax.lax.broadcasted_iota(jnp.int32, (N_NODES, E_EDGES), 0)
    iota_en = jax.lax.broadcasted_iota(jnp.int32, (E_EDGES, N_NODES), 1)
    Dh = (iota_ne == dst_row).astype(f32)
    Sh = (iota_ne == src_row).astype(f32)
    ShT = (iota_en == src_col).astype(f32)
    A = jnp.dot(Dh, ShT, preferred_element_type=f32)
    deg = jnp.sum(A, axis=1, keepdims=True)
    d31 = (dst_col == (N_NODES - 1)).astype(f32)
    a31 = jnp.dot(Sh, d31, preferred_element_type=f32)

    emb = emb_ref[0]
    wm1 = wmsg_ref[0, 0]
    wm2 = wmsg_ref[0, 1]
    ws1 = wself_ref[0, 0]
    ws2 = wself_ref[0, 1]
    Aemb = jnp.dot(A, emb, preferred_element_type=f32)
    bnode = (jnp.dot(Aemb, wm1, preferred_element_type=f32)
             + jnp.dot(emb, ws1, preferred_element_type=f32))

    proj = jnp.tanh(jnp.dot(x_ref[:], win_ref[0],
                            preferred_element_type=f32))
    P1 = jnp.dot(proj, wm1, preferred_element_type=f32)
    P2 = jnp.dot(proj, ws1, preferred_element_type=f32)

    g = jnp.zeros((N_F, D_HID), f32)
    for i in range(0, N_NODES, NODE_CHUNK):
        degk = deg[i:i + NODE_CHUNK]
        ak = a31[i:i + NODE_CHUNK]
        bk = bnode[i:i + NODE_CHUNK]
        h1k = jax.nn.relu(degk[:, :, None] * P1[None, :, :]
                          + P2[None, :, :] + bk[:, None, :])
        g = g + jnp.sum(h1k * ak[:, :, None], axis=0)
    s = jax.nn.relu(deg[N_NODES - 1, 0] * P1 + P2
                    + bnode[N_NODES - 1:N_NODES, :])
    enc = jax.nn.relu(jnp.dot(g, wm2, preferred_element_type=f32)
                      + jnp.dot(s, ws2, preferred_element_type=f32))

    logits_c = jnp.dot(enc, watt_ref[0], preferred_element_type=f32)
    v_c = jnp.dot(enc, wout_ref[0], preferred_element_type=f32)

    @pl.when(c == 0)
    def _():
        logits_ref[:] = logits_c
        v_ref[:] = v_c

    @pl.when(c != 0)
    def _():
        logits_ref[:] = logits_ref[:] + logits_c
        v_ref[:] = v_ref[:] + v_c


def _head_kernel(lg_ref, mask_ref, v_ref, bout_ref, out_ref):
    lg = jnp.where(mask_ref[:] > 0, lg_ref[:], jnp.float32(-1e9))
    m = jnp.max(lg, axis=1, keepdims=True)
    e = jnp.exp(lg - m)
    attn = e / jnp.sum(e, axis=1, keepdims=True)
    cols = []
    for j in range(C):
        vj = v_ref[:, :, j]
        cols.append(jnp.sum(attn * vj, axis=1, keepdims=True))
    out_ref[:] = jnp.concatenate(cols, axis=1) + bout_ref[:]


def kernel(sensor_seq, mask, node_emb, edge_src, edge_dst, W_in, W_msg,
           W_self, w_att, W_out, b_out):
    x = sensor_seq.reshape(N_F, D_IN)
    es = edge_src.astype(jnp.int32)
    ed = edge_dst.astype(jnp.int32)
    es_row = es.reshape(C, 1, E_EDGES)
    es_col = es.reshape(C, E_EDGES, 1)
    ed_row = ed.reshape(C, 1, E_EDGES)
    ed_col = ed.reshape(C, E_EDGES, 1)
    watt = w_att.reshape(C, D_HID, 1)
    wout = W_out.reshape(C, D_HID, C)

    logits, v = pl.pallas_call(
        _branch_kernel,
        grid=(C,),
        in_specs=[
            pl.BlockSpec((N_F, D_IN), lambda c: (0, 0)),
            pl.BlockSpec((1, N_NODES, D_HID), lambda c: (c, 0, 0)),
            pl.BlockSpec((1, 1, E_EDGES), lambda c: (c, 0, 0)),
            pl.BlockSpec((1, E_EDGES, 1), lambda c: (c, 0, 0)),
            pl.BlockSpec((1, 1, E_EDGES), lambda c: (c, 0, 0)),
            pl.BlockSpec((1, E_EDGES, 1), lambda c: (c, 0, 0)),
            pl.BlockSpec((1, D_IN, D_HID), lambda c: (c, 0, 0)),
            pl.BlockSpec((1, 2, D_HID, D_HID), lambda c: (c, 0, 0, 0)),
            pl.BlockSpec((1, 2, D_HID, D_HID), lambda c: (c, 0, 0, 0)),
            pl.BlockSpec((1, D_HID, 1), lambda c: (c, 0, 0)),
            pl.BlockSpec((1, D_HID, C), lambda c: (c, 0, 0)),
        ],
        out_specs=[
            pl.BlockSpec((N_F, 1), lambda c: (0, 0)),
            pl.BlockSpec((N_F, C), lambda c: (0, 0)),
        ],
        out_shape=[
            jax.ShapeDtypeStruct((N_F, 1), jnp.float32),
            jax.ShapeDtypeStruct((N_F, C), jnp.float32),
        ],
    )(x, node_emb, es_row, es_col, ed_row, ed_col, W_in, W_msg, W_self,
      watt, wout)

    lg3 = logits.reshape(B, T)
    v3 = v.reshape(B, T, C)
    out = pl.pallas_call(
        _head_kernel,
        out_shape=jax.ShapeDtypeStruct((B, C), jnp.float32),
    )(lg3, mask, v3, b_out.reshape(1, C))
    return out


# speedup vs baseline: 55.9392x; 1.0097x over previous
import functools

import jax
import jax.numpy as jnp
from jax.experimental import pallas as pl
from jax.experimental.pallas import tpu as pltpu

C = 8
N_NODES = 32
D_HID = 128
E_EDGES = 128
D_IN = 1024
B = 32
T = 30
N_F = B * T
NODE_CHUNK = 4


def _branch_kernel(x_ref, emb_ref, es_row_ref, es_col_ref, ed_row_ref,
                   ed_col_ref, win_ref, wmsg_ref, wself_ref, watt_ref,
                   wout_ref, logits_ref, v_ref, c1_ref, c2_ref, bp_ref):
    c = pl.program_id(0)
    f32 = jnp.float32

    src_row = es_row_ref[0]
    src_col = es_col_ref[0]
    dst_row = ed_row_ref[0]
    dst_col = ed_col_ref[0]

    iota_ne = jax.lax.broadcasted_iota(jnp.int32, (N_NODES, E_EDGES), 0)
    iota_en = jax.lax.broadcasted_iota(jnp.int32, (E_EDGES, N_NODES), 1)
    Dh = (iota_ne == dst_row).astype(f32)
    Sh = (iota_ne == src_row).astype(f32)
    ShT = (iota_en == src_col).astype(f32)
    A = jnp.dot(Dh, ShT, preferred_element_type=f32)
    deg = jnp.sum(A, axis=1, keepdims=True)
    d31 = (dst_col == (N_NODES - 1)).astype(f32)
    a31 = jnp.dot(Sh, d31, preferred_element_type=f32)

    emb = emb_ref[0]
    wm1 = wmsg_ref[0, 0]
    wm2 = wmsg_ref[0, 1]
    ws1 = wself_ref[0, 0]
    ws2 = wself_ref[0, 1]
    Aemb = jnp.dot(A, emb, preferred_element_type=f32)
    bnode = (jnp.dot(Aemb, wm1, preferred_element_type=f32)
             + jnp.dot(emb, ws1, preferred_element_type=f32))

    proj = jnp.tanh(jnp.dot(x_ref[:], win_ref[0],
                            preferred_element_type=f32))
    P1 = jnp.dot(proj, wm1, preferred_element_type=f32)
    P2 = jnp.dot(proj, ws1, preferred_element_type=f32)

    c1 = a31 * deg
    bp = a31 * bnode
    p = jnp.int32(0)
    for n in range(N_NODES):
        an = a31[n, 0]

        @pl.when(an > 0)
        def _(n=n, p=p):
            c1_ref[pl.ds(p, 1), :] = c1[n:n + 1, :]
            c2_ref[pl.ds(p, 1), :] = a31[n:n + 1, :]
            bp_ref[pl.ds(p, 1), :] = bp[n:n + 1, :]

        p = p + (an > 0).astype(jnp.int32)

    def body(i, g):
        c1i = c1_ref[pl.ds(i, 1), :]
        c2i = c2_ref[pl.ds(i, 1), :]
        bi = bp_ref[pl.ds(i, 1), :]
        return g + jax.nn.relu(c1i * P1 + c2i * P2 + bi)

    g = jax.lax.fori_loop(0, p, body, jnp.zeros((N_F, D_HID), f32))
    s = jax.nn.relu(deg[N_NODES - 1, 0] * P1 + P2
                    + bnode[N_NODES - 1:N_NODES, :])
    enc = jax.nn.relu(jnp.dot(g, wm2, preferred_element_type=f32)
                      + jnp.dot(s, ws2, preferred_element_type=f32))

    logits_c = jnp.dot(enc, watt_ref[0], preferred_element_type=f32)
    v_c = jnp.dot(enc, wout_ref[0], preferred_element_type=f32)

    @pl.when(c == 0)
    def _():
        logits_ref[:] = logits_c
        v_ref[:] = v_c

    @pl.when(c != 0)
    def _():
        logits_ref[:] = logits_ref[:] + logits_c
        v_ref[:] = v_ref[:] + v_c


def _head_kernel(lg_ref, mask_ref, v_ref, bout_ref, out_ref):
    lg = jnp.where(mask_ref[:] > 0, lg_ref[:], jnp.float32(-1e9))
    m = jnp.max(lg, axis=1, keepdims=True)
    e = jnp.exp(lg - m)
    attn = e / jnp.sum(e, axis=1, keepdims=True)
    cols = []
    for j in range(C):
        vj = v_ref[:, :, j]
        cols.append(jnp.sum(attn * vj, axis=1, keepdims=True))
    out_ref[:] = jnp.concatenate(cols, axis=1) + bout_ref[:]


def kernel(sensor_seq, mask, node_emb, edge_src, edge_dst, W_in, W_msg,
           W_self, w_att, W_out, b_out):
    x = sensor_seq.reshape(N_F, D_IN)
    es = edge_src.astype(jnp.int32)
    ed = edge_dst.astype(jnp.int32)
    es_row = es.reshape(C, 1, E_EDGES)
    es_col = es.reshape(C, E_EDGES, 1)
    ed_row = ed.reshape(C, 1, E_EDGES)
    ed_col = ed.reshape(C, E_EDGES, 1)
    watt = w_att.reshape(C, D_HID, 1)
    wout = W_out.reshape(C, D_HID, C)

    logits, v = pl.pallas_call(
        _branch_kernel,
        grid=(C,),
        in_specs=[
            pl.BlockSpec((N_F, D_IN), lambda c: (0, 0)),
            pl.BlockSpec((1, N_NODES, D_HID), lambda c: (c, 0, 0)),
            pl.BlockSpec((1, 1, E_EDGES), lambda c: (c, 0, 0)),
            pl.BlockSpec((1, E_EDGES, 1), lambda c: (c, 0, 0)),
            pl.BlockSpec((1, 1, E_EDGES), lambda c: (c, 0, 0)),
            pl.BlockSpec((1, E_EDGES, 1), lambda c: (c, 0, 0)),
            pl.BlockSpec((1, D_IN, D_HID), lambda c: (c, 0, 0)),
            pl.BlockSpec((1, 2, D_HID, D_HID), lambda c: (c, 0, 0, 0)),
            pl.BlockSpec((1, 2, D_HID, D_HID), lambda c: (c, 0, 0, 0)),
            pl.BlockSpec((1, D_HID, 1), lambda c: (c, 0, 0)),
            pl.BlockSpec((1, D_HID, C), lambda c: (c, 0, 0)),
        ],
        out_specs=[
            pl.BlockSpec((N_F, 1), lambda c: (0, 0)),
            pl.BlockSpec((N_F, C), lambda c: (0, 0)),
        ],
        out_shape=[
            jax.ShapeDtypeStruct((N_F, 1), jnp.float32),
            jax.ShapeDtypeStruct((N_F, C), jnp.float32),
        ],
        scratch_shapes=[
            pltpu.VMEM((N_NODES, 1), jnp.float32),
            pltpu.VMEM((N_NODES, 1), jnp.float32),
            pltpu.VMEM((N_NODES, D_HID), jnp.float32),
        ],
    )(x, node_emb, es_row, es_col, ed_row, ed_col, W_in, W_msg, W_self,
      watt, wout)

    lg3 = logits.reshape(B, T)
    v3 = v.reshape(B, T, C)
    out = pl.pallas_call(
        _head_kernel,
        out_shape=jax.ShapeDtypeStruct((B, C), jnp.float32),
    )(lg3, mask, v3, b_out.reshape(1, C))
    return out


# speedup vs baseline: 56.1203x; 1.0032x over previous
import jax
import jax.numpy as jnp
from jax.experimental import pallas as pl
from jax.experimental.pallas import tpu as pltpu

C = 8
N_NODES = 32
D_HID = 128
E_EDGES = 128
D_IN = 1024
B = 32
T = 30
N_F = B * T


def _branch_kernel(x_ref, emb_ref, es_row_ref, es_col_ref, ed_row_ref,
                   ed_col_ref, win_ref, wmsg_ref, wself_ref, watt_ref,
                   wout_ref, logits_ref, v_ref, c1_ref, c2_ref, bp_ref):
    f32 = jnp.float32
    iota_ne = jax.lax.broadcasted_iota(jnp.int32, (N_NODES, E_EDGES), 0)
    iota_en = jax.lax.broadcasted_iota(jnp.int32, (E_EDGES, N_NODES), 1)

    logits_acc = jnp.zeros((N_F, 1), f32)
    v_acc = jnp.zeros((N_F, C), f32)

    for c in range(C):
        src_row = es_row_ref[c]
        src_col = es_col_ref[c]
        dst_row = ed_row_ref[c]
        dst_col = ed_col_ref[c]

        Dh = (iota_ne == dst_row).astype(f32)
        Sh = (iota_ne == src_row).astype(f32)
        ShT = (iota_en == src_col).astype(f32)
        A = jnp.dot(Dh, ShT, preferred_element_type=f32)
        deg = jnp.sum(A, axis=1, keepdims=True)
        d31 = (dst_col == (N_NODES - 1)).astype(f32)
        a31 = jnp.dot(Sh, d31, preferred_element_type=f32)

        emb = emb_ref[c]
        wm1 = wmsg_ref[c, 0]
        wm2 = wmsg_ref[c, 1]
        ws1 = wself_ref[c, 0]
        ws2 = wself_ref[c, 1]
        Aemb = jnp.dot(A, emb, preferred_element_type=f32)
        bnode = (jnp.dot(Aemb, wm1, preferred_element_type=f32)
                 + jnp.dot(emb, ws1, preferred_element_type=f32))

        proj = jnp.tanh(jnp.dot(x_ref[:], win_ref[c],
                                preferred_element_type=f32))
        P1 = jnp.dot(proj, wm1, preferred_element_type=f32)
        P2 = jnp.dot(proj, ws1, preferred_element_type=f32)

        c1 = a31 * deg
        bp = a31 * bnode
        p = jnp.int32(0)
        for n in range(N_NODES):
            an = a31[n, 0]

            @pl.when(an > 0)
            def _(n=n, p=p):
                c1_ref[pl.ds(p, 1), :] = c1[n:n + 1, :]
                c2_ref[pl.ds(p, 1), :] = a31[n:n + 1, :]
                bp_ref[pl.ds(p, 1), :] = bp[n:n + 1, :]

            p = p + (an > 0).astype(jnp.int32)

        def body(i, g):
            c1i = c1_ref[pl.ds(i, 1), :]
            c2i = c2_ref[pl.ds(i, 1), :]
            bi = bp_ref[pl.ds(i, 1), :]
            return g + jax.nn.relu(c1i * P1 + c2i * P2 + bi)

        g = jax.lax.fori_loop(0, p, body, jnp.zeros((N_F, D_HID), f32))
        s = jax.nn.relu(deg[N_NODES - 1, 0] * P1 + P2
                        + bnode[N_NODES - 1:N_NODES, :])
        enc = jax.nn.relu(jnp.dot(g, wm2, preferred_element_type=f32)
                          + jnp.dot(s, ws2, preferred_element_type=f32))

        logits_acc = logits_acc + jnp.dot(enc, watt_ref[c],
                                          preferred_element_type=f32)
        v_acc = v_acc + jnp.dot(enc, wout_ref[c],
                                preferred_element_type=f32)

    logits_ref[:] = logits_acc
    v_ref[:] = v_acc


def _head_kernel(lg_ref, mask_ref, v_ref, bout_ref, out_ref):
    lg = jnp.where(mask_ref[:] > 0, lg_ref[:], jnp.float32(-1e9))
    m = jnp.max(lg, axis=1, keepdims=True)
    e = jnp.exp(lg - m)
    attn = e / jnp.sum(e, axis=1, keepdims=True)
    cols = []
    for j in range(C):
        vj = v_ref[:, :, j]
        cols.append(jnp.sum(attn * vj, axis=1, keepdims=True))
    out_ref[:] = jnp.concatenate(cols, axis=1) + bout_ref[:]


def kernel(sensor_seq, mask, node_emb, edge_src, edge_dst, W_in, W_msg,
           W_self, w_att, W_out, b_out):
    x = sensor_seq.reshape(N_F, D_IN)
    es = edge_src.astype(jnp.int32)
    ed = edge_dst.astype(jnp.int32)
    es_row = es.reshape(C, 1, E_EDGES)
    es_col = es.reshape(C, E_EDGES, 1)
    ed_row = ed.reshape(C, 1, E_EDGES)
    ed_col = ed.reshape(C, E_EDGES, 1)
    watt = w_att.reshape(C, D_HID, 1)
    wout = W_out.reshape(C, D_HID, C)

    logits, v = pl.pallas_call(
        _branch_kernel,
        out_shape=[
            jax.ShapeDtypeStruct((N_F, 1), jnp.float32),
            jax.ShapeDtypeStruct((N_F, C), jnp.float32),
        ],
        scratch_shapes=[
            pltpu.VMEM((N_NODES, 1), jnp.float32),
            pltpu.VMEM((N_NODES, 1), jnp.float32),
            pltpu.VMEM((N_NODES, D_HID), jnp.float32),
        ],
    )(x, node_emb, es_row, es_col, ed_row, ed_col, W_in, W_msg, W_self,
      watt, wout)

    lg3 = logits.reshape(B, T)
    v3 = v.reshape(B, T, C)
    out = pl.pallas_call(
        _head_kernel,
        out_shape=jax.ShapeDtypeStruct((B, C), jnp.float32),
    )(lg3, mask, v3, b_out.reshape(1, C))
    return out
